# Initial kernel scaffold; baseline (speedup 1.0000x reference)
import jax
import jax.numpy as jnp
from jax.experimental import pallas as pl


def kernel(inputs, row_embedding, col_embedding):
    raise NotImplementedError("write your pallas kernel here")


# speedup vs baseline: 1.1242x; 1.1242x over previous
import numpy as np
import jax
import jax.numpy as jnp
from jax.experimental import pallas as pl
from jax.experimental.pallas import tpu as pltpu

_PATCH = 16
_HEIGHT = 512
_WIDTH = 512
_DEPTH = 128
_EMBED = 768


def _axis_positions(axis_num):
    n = axis_num // _PATCH
    idx = np.arange(n, dtype=np.float64)
    frm = np.round(idx * _PATCH / axis_num * _DEPTH).astype(np.int32)
    to = np.round((idx + 1) * _PATCH / axis_num * _DEPTH).astype(np.int32)
    return np.round((frm + to).astype(np.float64) / 2.0).astype(np.int32)


_ROW_AXIS = _axis_positions(_HEIGHT)
_COL_AXIS = _axis_positions(_WIDTH)
_NROWS = _HEIGHT // _PATCH
_NCOLS = _WIDTH // _PATCH


def _add_kernel(x_ref, row_ref, col_ref, o_ref, enc_ref):
    @pl.when(pl.program_id(0) == 0)
    def _():
        row_rows = jnp.concatenate(
            [row_ref[int(p)][None, :] for p in _ROW_AXIS], axis=0
        )
        col_rows = jnp.concatenate(
            [col_ref[int(p)][None, :] for p in _COL_AXIS], axis=0
        )
        enc = row_rows[:, None, :] + col_rows[None, :, :]
        enc_ref[...] = enc.reshape(_NROWS * _NCOLS, _EMBED)

    o_ref[...] = x_ref[...] + enc_ref[...][None, :, :]


def kernel(inputs, row_embedding, col_embedding):
    B, S, E = inputs.shape
    return pl.pallas_call(
        _add_kernel,
        grid=(B,),
        in_specs=[
            pl.BlockSpec((1, S, E), lambda b: (b, 0, 0)),
            pl.BlockSpec((_DEPTH, E), lambda b: (0, 0)),
            pl.BlockSpec((_DEPTH, E), lambda b: (0, 0)),
        ],
        out_specs=pl.BlockSpec((1, S, E), lambda b: (b, 0, 0)),
        out_shape=jax.ShapeDtypeStruct((B, S, E), inputs.dtype),
        scratch_shapes=[pltpu.VMEM((S, E), jnp.float32)],
    )(inputs, row_embedding, col_embedding)


# speedup vs baseline: 1.1655x; 1.0368x over previous
import numpy as np
import jax
import jax.numpy as jnp
from jax.experimental import pallas as pl
from jax.experimental.pallas import tpu as pltpu

_PATCH = 16
_HEIGHT = 512
_WIDTH = 512
_DEPTH = 128
_EMBED = 768


def _axis_positions(axis_num):
    n = axis_num // _PATCH
    idx = np.arange(n, dtype=np.float64)
    frm = np.round(idx * _PATCH / axis_num * _DEPTH).astype(np.int32)
    to = np.round((idx + 1) * _PATCH / axis_num * _DEPTH).astype(np.int32)
    return np.round((frm + to).astype(np.float64) / 2.0).astype(np.int32)


_ROW_AXIS = _axis_positions(_HEIGHT)
_COL_AXIS = _axis_positions(_WIDTH)
_NROWS = _HEIGHT // _PATCH
_NCOLS = _WIDTH // _PATCH


def _add_kernel(x_ref, row_ref, col_ref, o_ref, enc_ref):
    @pl.when(pl.program_id(0) == 0)
    def _():
        row_rows = jnp.concatenate(
            [row_ref[int(p)][None, :] for p in _ROW_AXIS], axis=0
        )
        col_rows = jnp.concatenate(
            [col_ref[int(p)][None, :] for p in _COL_AXIS], axis=0
        )
        enc = row_rows[:, None, :] + col_rows[None, :, :]
        enc_ref[...] = enc.reshape(_NROWS * _NCOLS, _EMBED)

    o_ref[...] = x_ref[...] + enc_ref[...][None, :, :]


_BB = 2


def kernel(inputs, row_embedding, col_embedding):
    B, S, E = inputs.shape
    return pl.pallas_call(
        _add_kernel,
        grid=(B // _BB,),
        in_specs=[
            pl.BlockSpec((_BB, S, E), lambda b: (b, 0, 0)),
            pl.BlockSpec((_DEPTH, E), lambda b: (0, 0)),
            pl.BlockSpec((_DEPTH, E), lambda b: (0, 0)),
        ],
        out_specs=pl.BlockSpec((_BB, S, E), lambda b: (b, 0, 0)),
        out_shape=jax.ShapeDtypeStruct((B, S, E), inputs.dtype),
        scratch_shapes=[pltpu.VMEM((S, E), jnp.float32)],
    )(inputs, row_embedding, col_embedding)


# speedup vs baseline: 1.2053x; 1.0341x over previous
import numpy as np
import jax
import jax.numpy as jnp
from jax.experimental import pallas as pl
from jax.experimental.pallas import tpu as pltpu

_PATCH = 16
_HEIGHT = 512
_WIDTH = 512
_DEPTH = 128
_EMBED = 768


def _axis_positions(axis_num):
    n = axis_num // _PATCH
    idx = np.arange(n, dtype=np.float64)
    frm = np.round(idx * _PATCH / axis_num * _DEPTH).astype(np.int32)
    to = np.round((idx + 1) * _PATCH / axis_num * _DEPTH).astype(np.int32)
    return np.round((frm + to).astype(np.float64) / 2.0).astype(np.int32)


_ROW_AXIS = _axis_positions(_HEIGHT)
_COL_AXIS = _axis_positions(_WIDTH)
_NROWS = _HEIGHT // _PATCH
_NCOLS = _WIDTH // _PATCH


def _add_kernel(x_ref, row_ref, col_ref, o_ref, enc_ref):
    @pl.when(pl.program_id(0) == 0)
    def _():
        row_rows = jnp.concatenate(
            [row_ref[int(p)][None, :] for p in _ROW_AXIS], axis=0
        )
        col_rows = jnp.concatenate(
            [col_ref[int(p)][None, :] for p in _COL_AXIS], axis=0
        )
        enc = row_rows[:, None, :] + col_rows[None, :, :]
        enc_ref[...] = enc.reshape(_NROWS * _NCOLS, _EMBED)

    o_ref[...] = x_ref[...] + enc_ref[...][None, :, :]


_BB = 4


def kernel(inputs, row_embedding, col_embedding):
    B, S, E = inputs.shape
    return pl.pallas_call(
        _add_kernel,
        grid=(B // _BB,),
        in_specs=[
            pl.BlockSpec((_BB, S, E), lambda b: (b, 0, 0)),
            pl.BlockSpec((_DEPTH, E), lambda b: (0, 0)),
            pl.BlockSpec((_DEPTH, E), lambda b: (0, 0)),
        ],
        out_specs=pl.BlockSpec((_BB, S, E), lambda b: (b, 0, 0)),
        out_shape=jax.ShapeDtypeStruct((B, S, E), inputs.dtype),
        scratch_shapes=[pltpu.VMEM((S, E), jnp.float32)],
    )(inputs, row_embedding, col_embedding)
